# baseline (speedup 1.0000x reference)
import functools

import jax
import jax.numpy as jnp
from jax import lax
from jax.experimental import pallas as pl
from jax.experimental.pallas import tpu as pltpu
from jax.experimental.pallas import tpu_sc as plsc

N_USERS = 5000
M_ITEMS = 5000
N_NODES = N_USERS + M_ITEMS
N_EDGES = 320000
D = 128
N_LAYERS = 3

NC = 2
NS = 16
NW = NC * NS

CHUNK = 80
N_CHUNKS = 128
E_PER_TILE = CHUNK * N_CHUNKS
E_PAD = E_PER_TILE * NW
ZCH = 80
N_ZCH = N_NODES // ZCH


def _tc_prologue(user_emb, item_emb, creator_feat, item_feat, Wc, bc, Wi, bi, lw):
    def body(u_ref, i_ref, cf_ref, if_ref, wc_ref, bc_ref, wi_ref, bi_ref,
             lw_ref, emb_ref, acc_ref):
        u = u_ref[...] + jnp.dot(cf_ref[...], wc_ref[...],
                                 preferred_element_type=jnp.float32)
        u = u + bc_ref[...][None, :]
        it = i_ref[...] + jnp.dot(if_ref[...], wi_ref[...],
                                  preferred_element_type=jnp.float32)
        it = it + bi_ref[...][None, :]
        w0 = lw_ref[0]
        emb_ref[pl.ds(0, N_USERS), :] = u
        emb_ref[pl.ds(N_USERS, M_ITEMS), :] = it
        acc_ref[pl.ds(0, N_USERS), :] = u * w0
        acc_ref[pl.ds(N_USERS, M_ITEMS), :] = it * w0

    return pl.pallas_call(
        body,
        out_shape=(
            jax.ShapeDtypeStruct((N_NODES, D), jnp.float32),
            jax.ShapeDtypeStruct((N_NODES, D), jnp.float32),
        ),
        in_specs=[
            pl.BlockSpec(memory_space=pltpu.VMEM),
            pl.BlockSpec(memory_space=pltpu.VMEM),
            pl.BlockSpec(memory_space=pltpu.VMEM),
            pl.BlockSpec(memory_space=pltpu.VMEM),
            pl.BlockSpec(memory_space=pltpu.VMEM),
            pl.BlockSpec(memory_space=pltpu.VMEM),
            pl.BlockSpec(memory_space=pltpu.VMEM),
            pl.BlockSpec(memory_space=pltpu.VMEM),
            pl.BlockSpec(memory_space=pltpu.SMEM),
        ],
        out_specs=(
            pl.BlockSpec(memory_space=pltpu.VMEM),
            pl.BlockSpec(memory_space=pltpu.VMEM),
        ),
    )(user_emb, item_emb, creator_feat, item_feat, Wc, bc, Wi, bi, lw)


def _lane_broadcast(vvec, j):
    bidx = jnp.broadcast_to(j, (16,)).astype(jnp.int32)
    dnums = lax.GatherDimensionNumbers(
        offset_dims=(), collapsed_slice_dims=(0,), start_index_map=(0,))
    return lax.gather(vvec, bidx[:, None], dnums, slice_sizes=(1,),
                      mode=lax.GatherScatterMode.PROMISE_IN_BOUNDS)


def _sc_scatter(emb, packed):
    mesh = plsc.VectorSubcoreMesh(core_axis_name="c", subcore_axis_name="s")

    @functools.partial(
        pl.kernel,
        mesh=mesh,
        out_type=jax.ShapeDtypeStruct((NC, N_NODES, D), jnp.float32),
        scratch_types=[
            pltpu.VMEM_SHARED((N_NODES, D), jnp.float32),
            pltpu.VMEM((3, CHUNK), jnp.int32),
            pltpu.VMEM((3, CHUNK), jnp.int32),
            pltpu.VMEM((3, CHUNK), jnp.int32),
            pltpu.VMEM((3, CHUNK), jnp.int32),
            pltpu.VMEM((CHUNK, D), jnp.float32),
            pltpu.VMEM((CHUNK, D), jnp.float32),
            pltpu.VMEM((CHUNK, D), jnp.float32),
            pltpu.VMEM((CHUNK, D), jnp.float32),
            pltpu.SemaphoreType.DMA,
            pltpu.SemaphoreType.DMA,
            pltpu.SemaphoreType.DMA,
            pltpu.SemaphoreType.DMA,
            pltpu.SemaphoreType.DMA,
            pltpu.SemaphoreType.DMA,
            pltpu.SemaphoreType.DMA,
            pltpu.SemaphoreType.DMA,
        ],
    )
    def run(emb_hbm, packed_hbm, out_hbm,
            acc_sh, i0, i1, i2, i3, g0, g1, s0, s1,
            is0, is1, is2, is3, gsem0, gsem1, ssem0, ssem1):
        c = lax.axis_index("c")
        s = lax.axis_index("s")
        ibufs = (i0, i1, i2, i3)
        isems = (is0, is1, is2, is3)
        gbufs = (g0, g1)
        sbufs = (s0, s1)
        gsems = (gsem0, gsem1)
        ssems = (ssem0, ssem1)

        def zrow(r, _):
            def zcol(d, _):
                s0[r, pl.ds(d * 16, 16)] = jnp.zeros((16,), jnp.float32)
                return 0
            return lax.fori_loop(0, D // 16, zcol, 0)
        lax.fori_loop(0, ZCH, zrow, 0)

        def zcopy(k, _):
            idx = s + k * NS

            @pl.when(idx < N_ZCH)
            def _():
                pltpu.sync_copy(s0.at[pl.ds(0, ZCH)], acc_sh.at[pl.ds(idx * ZCH, ZCH)])
            return 0
        lax.fori_loop(0, (N_ZCH + NS - 1) // NS, zcopy, 0)

        tb = (c * NS + s) * N_CHUNKS
        for q in range(4):
            pltpu.async_copy(packed_hbm.at[tb + q], ibufs[q], isems[q])
        pltpu.make_async_copy(packed_hbm.at[tb], i0, is0).wait()
        plsc.subcore_barrier()
        pltpu.async_copy(emb_hbm.at[i0.at[0]], g0, gsem0)

        def step_body(st, _):
            for k in range(4):
                i = st * 4 + k
                b, b1, q, q1, qn = k % 2, (k + 1) % 2, k, (k + 1) % 4, (k + 2) % 4
                gb, sb, gsm, ssm = gbufs[b], sbufs[b], gsems[b], ssems[b]

                pltpu.make_async_copy(emb_hbm.at[ibufs[q].at[0]], gb, gsm).wait()

                @pl.when(i >= 2)
                def _():
                    pltpu.make_async_copy(
                        sb, acc_sh.at[ibufs[qn].at[1]], ssm).wait()

                @pl.when(jnp.logical_and(i >= 2, i + 2 < N_CHUNKS))
                def _():
                    pltpu.async_copy(packed_hbm.at[tb + i + 2], ibufs[qn],
                                     isems[qn])

                @pl.when(i + 1 < N_CHUNKS)
                def _():
                    pltpu.make_async_copy(packed_hbm.at[tb + i + 1],
                                          ibufs[q1], isems[q1]).wait()
                    pltpu.async_copy(emb_hbm.at[ibufs[q1].at[0]],
                                     gbufs[b1], gsems[b1])

                def scale_group(g, _):
                    ivec = ibufs[q][2, pl.ds(g * 16, 16)]
                    vvec = lax.bitcast_convert_type(ivec, jnp.float32)

                    def scale(j, _):
                        e = g * 16 + j
                        bval = _lane_broadcast(vvec, j)
                        for dd in range(D // 16):
                            sl = pl.ds(dd * 16, 16)
                            sb[e, sl] = gb[e, sl] * bval
                        return 0
                    return lax.fori_loop(0, 16, scale, 0)
                lax.fori_loop(0, CHUNK // 16, scale_group, 0)

                pltpu.async_copy(sb, acc_sh.at[ibufs[q].at[1]], ssm, add=True)
            return 0
        lax.fori_loop(0, N_CHUNKS // 4, step_body, 0)
        pltpu.make_async_copy(s0, acc_sh.at[i2.at[1]], ssem0).wait()
        pltpu.make_async_copy(s1, acc_sh.at[i3.at[1]], ssem1).wait()
        plsc.subcore_barrier()

        def outcopy(k, _):
            idx = s + k * NS

            @pl.when(idx < N_ZCH)
            def _():
                r0 = idx * ZCH
                pltpu.sync_copy(acc_sh.at[pl.ds(r0, ZCH)], s0.at[pl.ds(0, ZCH)])
                pltpu.sync_copy(s0.at[pl.ds(0, ZCH)], out_hbm.at[c, pl.ds(r0, ZCH)])
            return 0
        lax.fori_loop(0, (N_ZCH + NS - 1) // NS, outcopy, 0)

    return run(emb, packed)


CCH = 80
N_CCH = N_NODES // CCH


def _sc_combine(part, acc, wvec16):
    mesh = plsc.VectorSubcoreMesh(core_axis_name="c", subcore_axis_name="s")

    @functools.partial(
        pl.kernel,
        mesh=mesh,
        out_type=(
            jax.ShapeDtypeStruct((N_NODES, D), jnp.float32),
            jax.ShapeDtypeStruct((N_NODES, D), jnp.float32),
        ),
        scratch_types=[
            pltpu.VMEM((CCH, D), jnp.float32),
            pltpu.VMEM((CCH, D), jnp.float32),
            pltpu.VMEM((CCH, D), jnp.float32),
            pltpu.VMEM((16,), jnp.float32),
            pltpu.SemaphoreType.DMA,
        ],
    )
    def run(part_hbm, acc_hbm, w_hbm, emb_out, acc_out, p0v, p1v, av, wv, sem):
        c = lax.axis_index("c")
        s = lax.axis_index("s")
        wid = s * NC + c
        pltpu.sync_copy(w_hbm, wv)
        w = wv[...]

        def do_chunk(k, _):
            idx = wid + k * NW

            @pl.when(idx < N_CCH)
            def _():
                r0 = idx * CCH
                pltpu.sync_copy(part_hbm.at[0, pl.ds(r0, CCH)], p0v)
                pltpu.sync_copy(part_hbm.at[1, pl.ds(r0, CCH)], p1v)
                pltpu.sync_copy(acc_hbm.at[pl.ds(r0, CCH)], av)

                def rbody(r, _):
                    def dbody(d, _):
                        sl = pl.ds(d * 16, 16)
                        ne = p0v[r, sl] + p1v[r, sl]
                        p0v[r, sl] = ne
                        av[r, sl] = av[r, sl] + ne * w
                        return 0
                    return lax.fori_loop(0, D // 16, dbody, 0)
                lax.fori_loop(0, CCH, rbody, 0)

                pltpu.sync_copy(p0v, emb_out.at[pl.ds(r0, CCH)])
                pltpu.sync_copy(av, acc_out.at[pl.ds(r0, CCH)])
            return 0
        lax.fori_loop(0, (N_CCH + NW - 1) // NW, do_chunk, 0)

    return run(part, acc, wvec16)


def kernel(user_emb, item_emb, creator_feat, item_feat, Wc, bc, Wi, bi,
           adj_values, layer_weights, adj_indices):
    rows = adj_indices[0]
    cols = adj_indices[1]
    pad = E_PAD - N_EDGES
    pad_idx = (jnp.arange(pad, dtype=jnp.int32) * 13) % N_NODES
    rows_p = jnp.concatenate([rows, pad_idx]).reshape(E_PAD // CHUNK, CHUNK)
    cols_p = jnp.concatenate([cols, pad_idx]).reshape(E_PAD // CHUNK, CHUNK)
    vals_p = jnp.concatenate(
        [adj_values, jnp.zeros((pad,), jnp.float32)]).reshape(E_PAD // CHUNK, CHUNK)
    packed = jnp.stack(
        [cols_p, rows_p, lax.bitcast_convert_type(vals_p, jnp.int32)], axis=1)

    emb, acc = _tc_prologue(user_emb, item_emb, creator_feat, item_feat,
                            Wc, bc, Wi, bi, layer_weights)
    for l in range(1, N_LAYERS + 1):
        part = _sc_scatter(emb, packed)
        wvec16 = jnp.broadcast_to(layer_weights[l], (16,))
        emb, acc = _sc_combine(part, acc, wvec16)

    return acc[:N_USERS], acc[N_USERS:]


# speedup vs baseline: 1.6986x; 1.6986x over previous
import functools

import jax
import jax.numpy as jnp
from jax import lax
from jax.experimental import pallas as pl
from jax.experimental.pallas import tpu as pltpu
from jax.experimental.pallas import tpu_sc as plsc

N_USERS = 5000
M_ITEMS = 5000
N_NODES = N_USERS + M_ITEMS
N_EDGES = 320000
D = 128
N_LAYERS = 3

NC = 2
NS = 16
NW = NC * NS

CHUNK = 80
N_CHUNKS = 128
E_PER_TILE = CHUNK * N_CHUNKS
E_PAD = E_PER_TILE * NW
ZCH = 80
N_ZCH = N_NODES // ZCH


def _tc_prologue(user_emb, item_emb, creator_feat, item_feat, Wc, bc, Wi, bi, lw):
    def body(u_ref, i_ref, cf_ref, if_ref, wc_ref, bc_ref, wi_ref, bi_ref,
             lw_ref, emb_ref, acc_ref):
        u = u_ref[...] + jnp.dot(cf_ref[...], wc_ref[...],
                                 preferred_element_type=jnp.float32)
        u = u + bc_ref[...][None, :]
        it = i_ref[...] + jnp.dot(if_ref[...], wi_ref[...],
                                  preferred_element_type=jnp.float32)
        it = it + bi_ref[...][None, :]
        w0 = lw_ref[0]
        emb_ref[pl.ds(0, N_USERS), :] = u
        emb_ref[pl.ds(N_USERS, M_ITEMS), :] = it
        acc_ref[pl.ds(0, N_USERS), :] = u * w0
        acc_ref[pl.ds(N_USERS, M_ITEMS), :] = it * w0

    return pl.pallas_call(
        body,
        out_shape=(
            jax.ShapeDtypeStruct((N_NODES, D), jnp.float32),
            jax.ShapeDtypeStruct((N_NODES, D), jnp.float32),
        ),
        in_specs=[
            pl.BlockSpec(memory_space=pltpu.VMEM),
            pl.BlockSpec(memory_space=pltpu.VMEM),
            pl.BlockSpec(memory_space=pltpu.VMEM),
            pl.BlockSpec(memory_space=pltpu.VMEM),
            pl.BlockSpec(memory_space=pltpu.VMEM),
            pl.BlockSpec(memory_space=pltpu.VMEM),
            pl.BlockSpec(memory_space=pltpu.VMEM),
            pl.BlockSpec(memory_space=pltpu.VMEM),
            pl.BlockSpec(memory_space=pltpu.SMEM),
        ],
        out_specs=(
            pl.BlockSpec(memory_space=pltpu.VMEM),
            pl.BlockSpec(memory_space=pltpu.VMEM),
        ),
    )(user_emb, item_emb, creator_feat, item_feat, Wc, bc, Wi, bi, lw)


def _lane_broadcast(vvec, j):
    bidx = jnp.broadcast_to(j, (16,)).astype(jnp.int32)
    dnums = lax.GatherDimensionNumbers(
        offset_dims=(), collapsed_slice_dims=(0,), start_index_map=(0,))
    return lax.gather(vvec, bidx[:, None], dnums, slice_sizes=(1,),
                      mode=lax.GatherScatterMode.PROMISE_IN_BOUNDS)


def _sc_scatter(emb, packed):
    mesh = plsc.VectorSubcoreMesh(core_axis_name="c", subcore_axis_name="s")

    @functools.partial(
        pl.kernel,
        mesh=mesh,
        out_type=jax.ShapeDtypeStruct((NC, N_NODES, D), jnp.float32),
        scratch_types=[
            pltpu.VMEM_SHARED((N_NODES, D), jnp.float32),
            pltpu.VMEM((3, CHUNK), jnp.int32),
            pltpu.VMEM((3, CHUNK), jnp.int32),
            pltpu.VMEM((3, CHUNK), jnp.int32),
            pltpu.VMEM((3, CHUNK), jnp.int32),
            pltpu.VMEM((CHUNK, D), jnp.float32),
            pltpu.VMEM((CHUNK, D), jnp.float32),
            pltpu.VMEM((CHUNK, D), jnp.float32),
            pltpu.VMEM((CHUNK, D), jnp.float32),
            pltpu.SemaphoreType.DMA,
            pltpu.SemaphoreType.DMA,
            pltpu.SemaphoreType.DMA,
            pltpu.SemaphoreType.DMA,
            pltpu.SemaphoreType.DMA,
            pltpu.SemaphoreType.DMA,
            pltpu.SemaphoreType.DMA,
            pltpu.SemaphoreType.DMA,
        ],
    )
    def run(emb_hbm, packed_hbm, out_hbm,
            acc_sh, i0, i1, i2, i3, g0, g1, s0, s1,
            is0, is1, is2, is3, gsem0, gsem1, ssem0, ssem1):
        c = lax.axis_index("c")
        s = lax.axis_index("s")
        ibufs = (i0, i1, i2, i3)
        isems = (is0, is1, is2, is3)
        gbufs = (g0, g1)
        sbufs = (s0, s1)
        gsems = (gsem0, gsem1)
        ssems = (ssem0, ssem1)

        def zrow(r, _):
            def zcol(d, _):
                s0[r, pl.ds(d * 16, 16)] = jnp.zeros((16,), jnp.float32)
                return 0
            return lax.fori_loop(0, D // 16, zcol, 0)
        lax.fori_loop(0, ZCH, zrow, 0)

        def zcopy(k, _):
            idx = s + k * NS

            @pl.when(idx < N_ZCH)
            def _():
                pltpu.sync_copy(s0.at[pl.ds(0, ZCH)], acc_sh.at[pl.ds(idx * ZCH, ZCH)])
            return 0
        lax.fori_loop(0, (N_ZCH + NS - 1) // NS, zcopy, 0)

        tb = (c * NS + s) * N_CHUNKS
        for q in range(4):
            pltpu.async_copy(packed_hbm.at[tb + q], ibufs[q], isems[q])
        pltpu.make_async_copy(packed_hbm.at[tb], i0, is0).wait()
        plsc.subcore_barrier()
        pltpu.async_copy(emb_hbm.at[i0.at[0]], g0, gsem0)

        def step_body(st, _):
            for k in range(4):
                i = st * 4 + k
                b, b1, q, q1, qn = k % 2, (k + 1) % 2, k, (k + 1) % 4, (k + 2) % 4
                gb, sb, gsm, ssm = gbufs[b], sbufs[b], gsems[b], ssems[b]

                pltpu.make_async_copy(emb_hbm.at[ibufs[q].at[0]], gb, gsm).wait()

                @pl.when(i >= 2)
                def _():
                    pltpu.make_async_copy(
                        sb, acc_sh.at[ibufs[qn].at[1]], ssm).wait()

                @pl.when(jnp.logical_and(i >= 2, i + 2 < N_CHUNKS))
                def _():
                    pltpu.async_copy(packed_hbm.at[tb + i + 2], ibufs[qn],
                                     isems[qn])

                @pl.when(i + 1 < N_CHUNKS)
                def _():
                    pltpu.make_async_copy(packed_hbm.at[tb + i + 1],
                                          ibufs[q1], isems[q1]).wait()
                    pltpu.async_copy(emb_hbm.at[ibufs[q1].at[0]],
                                     gbufs[b1], gsems[b1])

                def scale_group(g, _):
                    ivec = ibufs[q][2, pl.ds(g * 16, 16)]
                    vvec = lax.bitcast_convert_type(ivec, jnp.float32)
                    bvals = [_lane_broadcast(vvec, j) for j in range(16)]
                    e0 = g * 16
                    for j in range(16):
                        for dd in range(D // 16):
                            sl = pl.ds(dd * 16, 16)
                            sb[e0 + j, sl] = gb[e0 + j, sl] * bvals[j]
                    return 0
                lax.fori_loop(0, CHUNK // 16, scale_group, 0)

                pltpu.async_copy(sb, acc_sh.at[ibufs[q].at[1]], ssm, add=True)
            return 0
        lax.fori_loop(0, N_CHUNKS // 4, step_body, 0)
        pltpu.make_async_copy(s0, acc_sh.at[i2.at[1]], ssem0).wait()
        pltpu.make_async_copy(s1, acc_sh.at[i3.at[1]], ssem1).wait()
        plsc.subcore_barrier()

        def outcopy(k, _):
            idx = s + k * NS

            @pl.when(idx < N_ZCH)
            def _():
                r0 = idx * ZCH
                pltpu.sync_copy(acc_sh.at[pl.ds(r0, ZCH)], s0.at[pl.ds(0, ZCH)])
                pltpu.sync_copy(s0.at[pl.ds(0, ZCH)], out_hbm.at[c, pl.ds(r0, ZCH)])
            return 0
        lax.fori_loop(0, (N_ZCH + NS - 1) // NS, outcopy, 0)

    return run(emb, packed)


CCH = 80
N_CCH = N_NODES // CCH


def _sc_combine(part, acc, wvec16):
    mesh = plsc.VectorSubcoreMesh(core_axis_name="c", subcore_axis_name="s")

    @functools.partial(
        pl.kernel,
        mesh=mesh,
        out_type=(
            jax.ShapeDtypeStruct((N_NODES, D), jnp.float32),
            jax.ShapeDtypeStruct((N_NODES, D), jnp.float32),
        ),
        scratch_types=[
            pltpu.VMEM((CCH, D), jnp.float32),
            pltpu.VMEM((CCH, D), jnp.float32),
            pltpu.VMEM((CCH, D), jnp.float32),
            pltpu.VMEM((16,), jnp.float32),
            pltpu.SemaphoreType.DMA,
        ],
    )
    def run(part_hbm, acc_hbm, w_hbm, emb_out, acc_out, p0v, p1v, av, wv, sem):
        c = lax.axis_index("c")
        s = lax.axis_index("s")
        wid = s * NC + c
        pltpu.sync_copy(w_hbm, wv)
        w = wv[...]

        def do_chunk(k, _):
            idx = wid + k * NW

            @pl.when(idx < N_CCH)
            def _():
                r0 = idx * CCH
                pltpu.sync_copy(part_hbm.at[0, pl.ds(r0, CCH)], p0v)
                pltpu.sync_copy(part_hbm.at[1, pl.ds(r0, CCH)], p1v)
                pltpu.sync_copy(acc_hbm.at[pl.ds(r0, CCH)], av)

                def rbody(r, _):
                    def dbody(d, _):
                        sl = pl.ds(d * 16, 16)
                        ne = p0v[r, sl] + p1v[r, sl]
                        p0v[r, sl] = ne
                        av[r, sl] = av[r, sl] + ne * w
                        return 0
                    return lax.fori_loop(0, D // 16, dbody, 0)
                lax.fori_loop(0, CCH, rbody, 0)

                pltpu.sync_copy(p0v, emb_out.at[pl.ds(r0, CCH)])
                pltpu.sync_copy(av, acc_out.at[pl.ds(r0, CCH)])
            return 0
        lax.fori_loop(0, (N_CCH + NW - 1) // NW, do_chunk, 0)

    return run(part, acc, wvec16)


def kernel(user_emb, item_emb, creator_feat, item_feat, Wc, bc, Wi, bi,
           adj_values, layer_weights, adj_indices):
    rows = adj_indices[0]
    cols = adj_indices[1]
    pad = E_PAD - N_EDGES
    pad_idx = (jnp.arange(pad, dtype=jnp.int32) * 13) % N_NODES
    rows_p = jnp.concatenate([rows, pad_idx]).reshape(E_PAD // CHUNK, CHUNK)
    cols_p = jnp.concatenate([cols, pad_idx]).reshape(E_PAD // CHUNK, CHUNK)
    vals_p = jnp.concatenate(
        [adj_values, jnp.zeros((pad,), jnp.float32)]).reshape(E_PAD // CHUNK, CHUNK)
    packed = jnp.stack(
        [cols_p, rows_p, lax.bitcast_convert_type(vals_p, jnp.int32)], axis=1)

    emb, acc = _tc_prologue(user_emb, item_emb, creator_feat, item_feat,
                            Wc, bc, Wi, bi, layer_weights)
    for l in range(1, N_LAYERS + 1):
        part = _sc_scatter(emb, packed)
        wvec16 = jnp.broadcast_to(layer_weights[l], (16,))
        emb, acc = _sc_combine(part, acc, wvec16)

    return acc[:N_USERS], acc[N_USERS:]
